# baseline (speedup 1.0000x reference)
import functools

import jax
import jax.numpy as jnp
from jax import lax
from jax.experimental import pallas as pl
from jax.experimental.pallas import tpu as pltpu
from jax.experimental.pallas import tpu_sc as plsc

_info = plsc.get_sparse_core_info()
_NC, _NS, _L = _info.num_cores, _info.num_subcores, _info.num_lanes
_NW = _NC * _NS


@functools.lru_cache(maxsize=None)
def _make_kernel(B, D, C):
    rows_per_w = B // _NW
    n_chunks = rows_per_w // C
    mesh = plsc.VectorSubcoreMesh(core_axis_name="c", subcore_axis_name="s")

    @functools.partial(
        pl.kernel,
        mesh=mesh,
        compiler_params=pltpu.CompilerParams(needs_layout_passes=False),
        out_type=[
            jax.ShapeDtypeStruct((B,), jnp.float32),
            jax.ShapeDtypeStruct((B,), jnp.float32),
        ],
        scratch_types=[
            pltpu.VMEM((rows_per_w,), jnp.int32),
            pltpu.VMEM((rows_per_w,), jnp.int32),
            pltpu.VMEM((rows_per_w,), jnp.int32),
            pltpu.VMEM((C, D), jnp.float32),
            pltpu.VMEM((C, D), jnp.float32),
            pltpu.VMEM((C, D), jnp.float32),
            pltpu.VMEM((C, D), jnp.float32),
            pltpu.VMEM((C, D), jnp.float32),
            pltpu.VMEM((C, D), jnp.float32),
            pltpu.VMEM((n_chunks, C), jnp.float32),
            pltpu.VMEM((n_chunks, C), jnp.float32),
            pltpu.SemaphoreType.DMA,
            pltpu.SemaphoreType.DMA,
            pltpu.SemaphoreType.DMA,
            pltpu.SemaphoreType.DMA,
        ],
    )
    def mf_kernel(user_h, item_i_h, item_j_h, eu_h, ei_h,
                  out_i_h, out_j_h,
                  uidx, iidx, jidx, eu0, ei0, ej0, eu1, ei1, ej1,
                  oi_v, oj_v, gsem0, gsem1, ssem, isem):
        wid = lax.axis_index("s") * _NC + lax.axis_index("c")
        base = wid * rows_per_w
        eu_s = (eu0, eu1)
        ei_s = (ei0, ei1)
        ej_s = (ej0, ej1)
        gsem = (gsem0, gsem1)
        lane = lax.iota(jnp.int32, _L)

        icp = (
            pltpu.async_copy(user_h.at[pl.ds(base, rows_per_w)], uidx, isem),
            pltpu.async_copy(item_i_h.at[pl.ds(base, rows_per_w)], iidx,
                             isem),
            pltpu.async_copy(item_j_h.at[pl.ds(base, rows_per_w)], jidx,
                             isem),
        )
        for cp in icp:
            cp.wait()

        n_groups = C // _L
        rows = [jnp.full((_L,), g * _L, jnp.int32) + lane
                for g in range(n_groups)]
        zeros = tuple(jnp.zeros((_L,), jnp.float32) for _ in range(n_groups))

        def compute_chunk(c, eu_v, ei_v, ej_v):
            def d_body(d, carry):
                accs_i = list(carry[0])
                accs_j = list(carry[1])
                offs = (lane + d) & (D - 1)
                for g in range(n_groups):
                    eu = plsc.load_gather(eu_v, [rows[g], offs])
                    ei = plsc.load_gather(ei_v, [rows[g], offs])
                    ej = plsc.load_gather(ej_v, [rows[g], offs])
                    accs_i[g] = accs_i[g] + eu * ei
                    accs_j[g] = accs_j[g] + eu * ej
                return tuple(accs_i), tuple(accs_j)

            accs_i, accs_j = lax.fori_loop(0, D, d_body, (zeros, zeros))
            for g in range(n_groups):
                oi_v[c, pl.ds(g * _L, _L)] = accs_i[g]
                oj_v[c, pl.ds(g * _L, _L)] = accs_j[g]
            return (
                pltpu.async_copy(
                    oi_v.at[c], out_i_h.at[pl.ds(base + c * C, C)], ssem),
                pltpu.async_copy(
                    oj_v.at[c], out_j_h.at[pl.ds(base + c * C, C)], ssem),
            )

        def fire(c, slot):
            return (
                pltpu.async_copy(eu_h.at[uidx.at[pl.ds(c * C, C)]],
                                 eu_s[slot], gsem[slot]),
                pltpu.async_copy(ei_h.at[iidx.at[pl.ds(c * C, C)]],
                                 ei_s[slot], gsem[slot]),
                pltpu.async_copy(ei_h.at[jidx.at[pl.ds(c * C, C)]],
                                 ej_s[slot], gsem[slot]),
            )

        def pair_body(p, carry):
            c0 = p * 2
            cp0 = fire(c0, 0)
            cp1 = fire(c0 + 1, 1)
            for cp in cp0:
                cp.wait()
            s0 = compute_chunk(c0, eu0, ei0, ej0)
            for cp in cp1:
                cp.wait()
            s1 = compute_chunk(c0 + 1, eu1, ei1, ej1)
            for cp in s0 + s1:
                cp.wait()
            return carry

        lax.fori_loop(0, n_chunks // 2, pair_body, 0)

    return mf_kernel

def kernel(user, item_i, item_j, embed_user, embed_item):
    B = user.shape[0]
    D = embed_user.shape[1]
    k = _make_kernel(B, D, 64)
    out_i, out_j = k(user.astype(jnp.int32), item_i.astype(jnp.int32),
                     item_j.astype(jnp.int32), embed_user, embed_item)
    return (out_i, out_j)


# speedup vs baseline: 1.0317x; 1.0317x over previous
import functools

import jax
import jax.numpy as jnp
from jax import lax
from jax.experimental import pallas as pl
from jax.experimental.pallas import tpu as pltpu
from jax.experimental.pallas import tpu_sc as plsc

_info = plsc.get_sparse_core_info()
_NC, _NS, _L = _info.num_cores, _info.num_subcores, _info.num_lanes
_NW = _NC * _NS


@functools.lru_cache(maxsize=None)
def _make_kernel(B, D, C):
    rows_per_w = B // _NW
    n_chunks = rows_per_w // C
    mesh = plsc.VectorSubcoreMesh(core_axis_name="c", subcore_axis_name="s")

    @functools.partial(
        pl.kernel,
        mesh=mesh,
        compiler_params=pltpu.CompilerParams(needs_layout_passes=False),
        out_type=[
            jax.ShapeDtypeStruct((B,), jnp.float32),
            jax.ShapeDtypeStruct((B,), jnp.float32),
        ],
        scratch_types=[
            pltpu.VMEM((rows_per_w,), jnp.int32),
            pltpu.VMEM((rows_per_w,), jnp.int32),
            pltpu.VMEM((rows_per_w,), jnp.int32),
            pltpu.VMEM((C, D), jnp.float32),
            pltpu.VMEM((C, D), jnp.float32),
            pltpu.VMEM((C, D), jnp.float32),
            pltpu.VMEM((C, D), jnp.float32),
            pltpu.VMEM((C, D), jnp.float32),
            pltpu.VMEM((C, D), jnp.float32),
            pltpu.VMEM((n_chunks, C), jnp.float32),
            pltpu.VMEM((n_chunks, C), jnp.float32),
            pltpu.SemaphoreType.DMA,
            pltpu.SemaphoreType.DMA,
            pltpu.SemaphoreType.DMA,
            pltpu.SemaphoreType.DMA,
        ],
    )
    def mf_kernel(user_h, item_i_h, item_j_h, eu_h, ei_h,
                  out_i_h, out_j_h,
                  uidx, iidx, jidx, eu0, ei0, ej0, eu1, ei1, ej1,
                  oi_v, oj_v, gsem0, gsem1, ssem, isem):
        wid = lax.axis_index("s") * _NC + lax.axis_index("c")
        base = wid * rows_per_w
        eu_s = (eu0, eu1)
        ei_s = (ei0, ei1)
        ej_s = (ej0, ej1)
        gsem = (gsem0, gsem1)
        lane = lax.iota(jnp.int32, _L)

        icp = (
            pltpu.async_copy(user_h.at[pl.ds(base, rows_per_w)], uidx, isem),
            pltpu.async_copy(item_i_h.at[pl.ds(base, rows_per_w)], iidx,
                             isem),
            pltpu.async_copy(item_j_h.at[pl.ds(base, rows_per_w)], jidx,
                             isem),
        )
        for cp in icp:
            cp.wait()

        n_groups = C // _L
        rows = [jnp.full((_L,), g * _L, jnp.int32) + lane
                for g in range(n_groups)]
        zeros = tuple(jnp.zeros((_L,), jnp.float32) for _ in range(n_groups))

        def compute_chunk(c, eu_v, ei_v, ej_v):
            def d_body(d, carry):
                accs_i = list(carry[0])
                accs_j = list(carry[1])
                offs = (lane + d) & (D - 1)
                for g in range(n_groups):
                    eu = plsc.load_gather(eu_v, [rows[g], offs])
                    ei = plsc.load_gather(ei_v, [rows[g], offs])
                    ej = plsc.load_gather(ej_v, [rows[g], offs])
                    accs_i[g] = accs_i[g] + eu * ei
                    accs_j[g] = accs_j[g] + eu * ej
                return tuple(accs_i), tuple(accs_j)

            accs_i, accs_j = lax.fori_loop(0, D, d_body, (zeros, zeros),
                                           unroll=2)
            for g in range(n_groups):
                oi_v[c, pl.ds(g * _L, _L)] = accs_i[g]
                oj_v[c, pl.ds(g * _L, _L)] = accs_j[g]
            return (
                pltpu.async_copy(
                    oi_v.at[c], out_i_h.at[pl.ds(base + c * C, C)], ssem),
                pltpu.async_copy(
                    oj_v.at[c], out_j_h.at[pl.ds(base + c * C, C)], ssem),
            )

        def fire(c, slot):
            return (
                pltpu.async_copy(eu_h.at[uidx.at[pl.ds(c * C, C)]],
                                 eu_s[slot], gsem[slot]),
                pltpu.async_copy(ei_h.at[iidx.at[pl.ds(c * C, C)]],
                                 ei_s[slot], gsem[slot]),
                pltpu.async_copy(ei_h.at[jidx.at[pl.ds(c * C, C)]],
                                 ej_s[slot], gsem[slot]),
            )

        def pair_body(p, carry):
            c0 = p * 2
            cp0 = fire(c0, 0)
            cp1 = fire(c0 + 1, 1)
            for cp in cp0:
                cp.wait()
            s0 = compute_chunk(c0, eu0, ei0, ej0)
            for cp in cp1:
                cp.wait()
            s1 = compute_chunk(c0 + 1, eu1, ei1, ej1)
            for cp in s0 + s1:
                cp.wait()
            return carry

        lax.fori_loop(0, n_chunks // 2, pair_body, 0)

    return mf_kernel

def kernel(user, item_i, item_j, embed_user, embed_item):
    B = user.shape[0]
    D = embed_user.shape[1]
    k = _make_kernel(B, D, 128)
    out_i, out_j = k(user.astype(jnp.int32), item_i.astype(jnp.int32),
                     item_j.astype(jnp.int32), embed_user, embed_item)
    return (out_i, out_j)


# speedup vs baseline: 1.0340x; 1.0023x over previous
import functools

import jax
import jax.numpy as jnp
from jax import lax
from jax.experimental import pallas as pl
from jax.experimental.pallas import tpu as pltpu
from jax.experimental.pallas import tpu_sc as plsc

_info = plsc.get_sparse_core_info()
_NC, _NS, _L = _info.num_cores, _info.num_subcores, _info.num_lanes
_NW = _NC * _NS


@functools.lru_cache(maxsize=None)
def _make_kernel(B, D, C):
    rows_per_w = B // _NW
    n_chunks = rows_per_w // C
    mesh = plsc.VectorSubcoreMesh(core_axis_name="c", subcore_axis_name="s")

    @functools.partial(
        pl.kernel,
        mesh=mesh,
        compiler_params=pltpu.CompilerParams(
            needs_layout_passes=False,
            disable_bounds_checks=True,
            disable_semaphore_checks=True,
        ),
        out_type=[
            jax.ShapeDtypeStruct((B,), jnp.float32),
            jax.ShapeDtypeStruct((B,), jnp.float32),
        ],
        scratch_types=[
            pltpu.VMEM((rows_per_w,), jnp.int32),
            pltpu.VMEM((rows_per_w,), jnp.int32),
            pltpu.VMEM((rows_per_w,), jnp.int32),
            pltpu.VMEM((C, D), jnp.float32),
            pltpu.VMEM((C, D), jnp.float32),
            pltpu.VMEM((C, D), jnp.float32),
            pltpu.VMEM((C, D), jnp.float32),
            pltpu.VMEM((C, D), jnp.float32),
            pltpu.VMEM((C, D), jnp.float32),
            pltpu.VMEM((n_chunks, C), jnp.float32),
            pltpu.VMEM((n_chunks, C), jnp.float32),
            pltpu.SemaphoreType.DMA,
            pltpu.SemaphoreType.DMA,
            pltpu.SemaphoreType.DMA,
            pltpu.SemaphoreType.DMA,
        ],
    )
    def mf_kernel(user_h, item_i_h, item_j_h, eu_h, ei_h,
                  out_i_h, out_j_h,
                  uidx, iidx, jidx, eu0, ei0, ej0, eu1, ei1, ej1,
                  oi_v, oj_v, gsem0, gsem1, ssem, isem):
        wid = lax.axis_index("s") * _NC + lax.axis_index("c")
        base = wid * rows_per_w
        eu_s = (eu0, eu1)
        ei_s = (ei0, ei1)
        ej_s = (ej0, ej1)
        gsem = (gsem0, gsem1)
        lane = lax.iota(jnp.int32, _L)

        icp = (
            pltpu.async_copy(user_h.at[pl.ds(base, rows_per_w)], uidx, isem),
            pltpu.async_copy(item_i_h.at[pl.ds(base, rows_per_w)], iidx,
                             isem),
            pltpu.async_copy(item_j_h.at[pl.ds(base, rows_per_w)], jidx,
                             isem),
        )
        for cp in icp:
            cp.wait()

        n_groups = C // _L
        rows = [jnp.full((_L,), g * _L, jnp.int32) + lane
                for g in range(n_groups)]
        zeros = tuple(jnp.zeros((_L,), jnp.float32) for _ in range(n_groups))

        def compute_chunk(c, eu_v, ei_v, ej_v):
            def d_body(d, carry):
                accs_i = list(carry[0])
                accs_j = list(carry[1])
                offs = (lane + d) & (D - 1)
                for g in range(n_groups):
                    eu = plsc.load_gather(eu_v, [rows[g], offs])
                    ei = plsc.load_gather(ei_v, [rows[g], offs])
                    ej = plsc.load_gather(ej_v, [rows[g], offs])
                    accs_i[g] = accs_i[g] + eu * ei
                    accs_j[g] = accs_j[g] + eu * ej
                return tuple(accs_i), tuple(accs_j)

            accs_i, accs_j = lax.fori_loop(0, D, d_body, (zeros, zeros))
            for g in range(n_groups):
                oi_v[c, pl.ds(g * _L, _L)] = accs_i[g]
                oj_v[c, pl.ds(g * _L, _L)] = accs_j[g]
            return (
                pltpu.async_copy(
                    oi_v.at[c], out_i_h.at[pl.ds(base + c * C, C)], ssem),
                pltpu.async_copy(
                    oj_v.at[c], out_j_h.at[pl.ds(base + c * C, C)], ssem),
            )

        def fire(c, slot):
            return (
                pltpu.async_copy(eu_h.at[uidx.at[pl.ds(c * C, C)]],
                                 eu_s[slot], gsem[slot]),
                pltpu.async_copy(ei_h.at[iidx.at[pl.ds(c * C, C)]],
                                 ei_s[slot], gsem[slot]),
                pltpu.async_copy(ei_h.at[jidx.at[pl.ds(c * C, C)]],
                                 ej_s[slot], gsem[slot]),
            )

        def pair_body(p, carry):
            c0 = p * 2
            cp0 = fire(c0, 0)
            cp1 = fire(c0 + 1, 1)
            for cp in cp0:
                cp.wait()
            s0 = compute_chunk(c0, eu0, ei0, ej0)
            for cp in cp1:
                cp.wait()
            s1 = compute_chunk(c0 + 1, eu1, ei1, ej1)
            for cp in s0 + s1:
                cp.wait()
            return carry

        lax.fori_loop(0, n_chunks // 2, pair_body, 0)

    return mf_kernel

def kernel(user, item_i, item_j, embed_user, embed_item):
    B = user.shape[0]
    D = embed_user.shape[1]
    k = _make_kernel(B, D, 128)
    out_i, out_j = k(user.astype(jnp.int32), item_i.astype(jnp.int32),
                     item_j.astype(jnp.int32), embed_user, embed_item)
    return (out_i, out_j)


# speedup vs baseline: 1.0572x; 1.0224x over previous
import functools

import jax
import jax.numpy as jnp
from jax import lax
from jax.experimental import pallas as pl
from jax.experimental.pallas import tpu as pltpu
from jax.experimental.pallas import tpu_sc as plsc

_info = plsc.get_sparse_core_info()
_NC, _NS, _L = _info.num_cores, _info.num_subcores, _info.num_lanes
_NW = _NC * _NS


@functools.lru_cache(maxsize=None)
def _make_kernel(B, D, C):
    rows_per_w = B // _NW
    n_chunks = rows_per_w // C
    mesh = plsc.VectorSubcoreMesh(core_axis_name="c", subcore_axis_name="s")

    @functools.partial(
        pl.kernel,
        mesh=mesh,
        compiler_params=pltpu.CompilerParams(needs_layout_passes=False),
        out_type=[
            jax.ShapeDtypeStruct((B,), jnp.float32),
            jax.ShapeDtypeStruct((B,), jnp.float32),
        ],
        scratch_types=[
            pltpu.VMEM((n_chunks, C), jnp.int32),
            pltpu.VMEM((n_chunks, C), jnp.int32),
            pltpu.VMEM((n_chunks, C), jnp.int32),
            pltpu.VMEM((C, D), jnp.float32),
            pltpu.VMEM((C, D), jnp.float32),
            pltpu.VMEM((C, D), jnp.float32),
            pltpu.VMEM((C, D), jnp.float32),
            pltpu.VMEM((C, D), jnp.float32),
            pltpu.VMEM((C, D), jnp.float32),
            pltpu.VMEM((n_chunks, C), jnp.float32),
            pltpu.VMEM((n_chunks, C), jnp.float32),
            pltpu.SemaphoreType.DMA,
            pltpu.SemaphoreType.DMA,
            pltpu.SemaphoreType.DMA,
        ],
    )
    def mf_kernel(user_h, item_i_h, item_j_h, eu_h, ei_h,
                  out_i_h, out_j_h,
                  uidx, iidx, jidx, eu0, ei0, ej0, eu1, ei1, ej1,
                  oi_v, oj_v, gsem0, gsem1, ssem):
        wid = lax.axis_index("s") * _NC + lax.axis_index("c")
        base = wid * rows_per_w
        eu_s = (eu0, eu1)
        ei_s = (ei0, ei1)
        ej_s = (ej0, ej1)
        gsem = (gsem0, gsem1)
        lane = lax.iota(jnp.int32, _L)

        n_groups = C // _L
        rows = [jnp.full((_L,), g * _L, jnp.int32) + lane
                for g in range(n_groups)]
        zeros = tuple(jnp.zeros((_L,), jnp.float32) for _ in range(n_groups))

        def compute_chunk(c, eu_v, ei_v, ej_v):
            def d_body(d, carry):
                accs_i = list(carry[0])
                accs_j = list(carry[1])
                offs = (lane + d) & (D - 1)
                for g in range(n_groups):
                    eu = plsc.load_gather(eu_v, [rows[g], offs])
                    ei = plsc.load_gather(ei_v, [rows[g], offs])
                    ej = plsc.load_gather(ej_v, [rows[g], offs])
                    accs_i[g] = accs_i[g] + eu * ei
                    accs_j[g] = accs_j[g] + eu * ej
                return tuple(accs_i), tuple(accs_j)

            accs_i, accs_j = lax.fori_loop(0, D, d_body, (zeros, zeros))
            for g in range(n_groups):
                oi_v[c, pl.ds(g * _L, _L)] = accs_i[g]
                oj_v[c, pl.ds(g * _L, _L)] = accs_j[g]
            return (
                pltpu.async_copy(
                    oi_v.at[c], out_i_h.at[pl.ds(base + c * C, C)], ssem),
                pltpu.async_copy(
                    oj_v.at[c], out_j_h.at[pl.ds(base + c * C, C)], ssem),
            )

        def stage_and_fire(c, slot):
            pltpu.sync_copy(user_h.at[pl.ds(base + c * C, C)], uidx.at[slot])
            pltpu.sync_copy(item_i_h.at[pl.ds(base + c * C, C)],
                            iidx.at[slot])
            pltpu.sync_copy(item_j_h.at[pl.ds(base + c * C, C)],
                            jidx.at[slot])
            return (
                pltpu.async_copy(eu_h.at[uidx.at[slot]], eu_s[slot],
                                 gsem[slot]),
                pltpu.async_copy(ei_h.at[iidx.at[slot]], ei_s[slot],
                                 gsem[slot]),
                pltpu.async_copy(ei_h.at[jidx.at[slot]], ej_s[slot],
                                 gsem[slot]),
            )

        def pair_body(p, carry):
            c0 = p * 2
            cp0 = stage_and_fire(c0, 0)
            cp1 = stage_and_fire(c0 + 1, 1)
            for cp in cp0:
                cp.wait()
            s0 = compute_chunk(c0, eu0, ei0, ej0)
            for cp in cp1:
                cp.wait()
            s1 = compute_chunk(c0 + 1, eu1, ei1, ej1)
            for cp in s0 + s1:
                cp.wait()
            return carry

        lax.fori_loop(0, n_chunks // 2, pair_body, 0)

    return mf_kernel

def kernel(user, item_i, item_j, embed_user, embed_item):
    B = user.shape[0]
    D = embed_user.shape[1]
    k = _make_kernel(B, D, 128)
    out_i, out_j = k(user.astype(jnp.int32), item_i.astype(jnp.int32),
                     item_j.astype(jnp.int32), embed_user, embed_item)
    return (out_i, out_j)
